# baseline (speedup 1.0000x reference)
import functools

import jax
import jax.numpy as jnp
from jax import lax
from jax.experimental import pallas as pl
from jax.experimental.pallas import tpu as pltpu
from jax.experimental.pallas import tpu_sc as plsc

N = 10000
E = 320000
D = 128
FQ = 96
NTILES = 16
EPT = E // NTILES
B = 80
CPT = EPT // B
RING = 5
MRING = RING + 1
NPAD = 10240
RPT = NPAD // NTILES

_RANGES = {
    (0, 0): ((0, 96, 1),),
    (0, 1): ((0, 32, 1), (32, 96, 2)),
    (1, 0): ((0, 64, 2), (64, 96, 3)),
    (1, 1): ((0, 96, 3),),
}


def _mm_body(x_ref, w_ref, b_ref, o0_ref, o1_ref, o2_ref, o3_ref):
    acc = (jnp.dot(x_ref[...], w_ref[...], preferred_element_type=jnp.float32)
           + b_ref[...]).astype(jnp.bfloat16)
    o0_ref[...] = acc[:, 0 * FQ:1 * FQ]
    o1_ref[...] = acc[:, 1 * FQ:2 * FQ]
    o2_ref[...] = acc[:, 2 * FQ:3 * FQ]
    o3_ref[...] = acc[:, 3 * FQ:4 * FQ]


def _matmul(x, w, b):
    R = 2000
    return pl.pallas_call(
        _mm_body,
        grid=(N // R,),
        in_specs=[
            pl.BlockSpec((R, D), lambda i: (i, 0)),
            pl.BlockSpec((D, 3 * D), lambda i: (0, 0)),
            pl.BlockSpec((1, 3 * D), lambda i: (0, 0)),
        ],
        out_specs=[pl.BlockSpec((R, FQ), lambda i: (i, 0))] * 4,
        out_shape=[jax.ShapeDtypeStruct((N, FQ), jnp.bfloat16)] * 4,
    )(x, w, b)


def _scale_group(rows_ref, cv, p, base, w1):
    ranges = _RANGES[(cv, p)]
    pw = {1: w1}
    if any(e >= 2 for (_, _, e) in ranges):
        pw[2] = w1 * w1
    if any(e == 3 for (_, _, e) in ranges):
        pw[3] = pw[2] * w1 if 2 in pw else w1 * w1 * w1
    iota = lax.iota(jnp.int32, 16)
    ridx = iota + base
    rows_bf, rows_f = rows_ref

    for (lo, hi, e) in ranges:
        mul = pw[e]

        @plsc.parallel_loop(lo // 2, hi // 2, unroll=16)
        def _(f):
            wcol = jnp.bitwise_xor(f, iota)
            wd = plsc.load_gather(rows_bf, [ridx, wcol])
            ab = plsc.bitcast(wd, jnp.bfloat16)
            a, b = plsc.unpack(ab, format=plsc.PackFormat.INTERLEAVED,
                               preferred_element_type=jnp.float32)
            plsc.store_scatter(rows_f, [ridx, wcol], a * mul)
            plsc.store_scatter(rows_f, [ridx, wcol + FQ // 2], b * mul)


def _drain(sem, src_like, dst_like):
    pltpu.make_async_copy(src_like, dst_like, sem).wait()


def _run_pass(cv, p, s, table, out, acc_dummy_hbm, meta_hbm, meta_v, rows,
              acc_sh, isem, gsem, ssem):
    cbase = s * CPT
    rows_bf, rows_f = rows
    meta_dummy = meta_hbm.at[0]
    gat_dummy = table.at[pl.ds(0, B)]
    sca_dummy = acc_dummy_hbm.at[pl.ds(0, B)]

    pltpu.async_copy(meta_hbm.at[cbase], meta_v.at[0], isem)
    pltpu.async_copy(meta_hbm.at[cbase + 1], meta_v.at[1], isem)
    _drain(isem, meta_dummy, meta_v.at[0])
    pltpu.async_copy(table.at[meta_v.at[0, 0]], rows_bf.at[pl.ds(0, B)], gsem)

    @pl.loop(0, CPT)
    def _it(i):
        slot = lax.rem(i, RING)
        mslot = lax.rem(i, MRING)

        @pl.when(i >= RING - 1)
        def _():
            _drain(ssem, sca_dummy, rows_f.at[pl.ds(0, B)])

        @pl.when(i + 2 < CPT)
        def _():
            pltpu.async_copy(meta_hbm.at[cbase + i + 2],
                             meta_v.at[lax.rem(i + 2, MRING)], isem)

        @pl.when(i + 1 < CPT)
        def _():
            _drain(isem, meta_dummy, meta_v.at[0])
            pltpu.async_copy(
                table.at[meta_v.at[lax.rem(i + 1, MRING), 0]],
                rows_bf.at[pl.ds(lax.rem(i + 1, RING) * B, B)], gsem)

        _drain(gsem, gat_dummy, rows_bf.at[pl.ds(0, B)])

        @plsc.parallel_loop(0, B // 16)
        def _grp(g):
            w1 = plsc.bitcast(meta_v[mslot, 2, pl.ds(g * 16, 16)],
                              jnp.float32)
            _scale_group((rows_bf, rows_f), cv, p, slot * B + g * 16, w1)

        pltpu.async_copy(rows_f.at[pl.ds(slot * B, B)],
                         acc_sh.at[meta_v.at[mslot, 1]], ssem, add=True)

    for _ in range(RING - 1):
        _drain(ssem, sca_dummy, rows_f.at[pl.ds(0, B)])


def _sc_body(q0, q1, q2, q3, meta_hbm,
             o00, o01, o10, o11,
             meta_v, rows_bf, rows_f, acc_sh, isem, gsem, ssem):
    c = lax.axis_index("c")
    s = lax.axis_index("s")
    tables = ((q0, q1), (q2, q3))
    outs = ((o00, o01), (o10, o11))

    for p in range(2):
        zero16 = jnp.zeros((16,), jnp.float32)

        @pl.loop(0, B)
        def _zrow(i):
            for j in range(FQ // 16):
                rows_f[i, pl.ds(j * 16, 16)] = zero16

        for k in range(RPT // B):
            pltpu.sync_copy(rows_f.at[pl.ds(0, B)],
                            acc_sh.at[pl.ds(s * RPT + k * B, B)])
        plsc.subcore_barrier()

        for cv in range(2):
            @pl.when(c == cv)
            def _():
                _run_pass(cv, p, s, tables[cv][p], outs[cv][p], o00,
                          meta_hbm, meta_v, (rows_bf, rows_f), acc_sh,
                          isem, gsem, ssem)

        plsc.subcore_barrier()

        for cv in range(2):
            @pl.when(c == cv)
            def _():
                for k in range(RPT // B):
                    r0 = s * RPT + k * B
                    pltpu.sync_copy(acc_sh.at[pl.ds(r0, B)],
                                    outs[cv][p].at[pl.ds(r0, B)])


_sc_scatter = functools.partial(
    pl.kernel,
    out_type=[jax.ShapeDtypeStruct((NPAD, FQ), jnp.float32)] * 4,
    mesh=plsc.VectorSubcoreMesh(core_axis_name="c", subcore_axis_name="s"),
    compiler_params=pltpu.CompilerParams(
        use_tc_tiling_on_sc=False, needs_layout_passes=False),
    scratch_types=[
        pltpu.VMEM((MRING, 3, B), jnp.int32),
        pltpu.VMEM((RING * B, FQ // 2), jnp.int32),
        pltpu.VMEM((RING * B, FQ), jnp.float32),
        pltpu.VMEM_SHARED((NPAD, FQ), jnp.float32),
        pltpu.SemaphoreType.DMA,
        pltpu.SemaphoreType.DMA,
        pltpu.SemaphoreType.DMA,
    ],
)(_sc_body)


def _ep_body(a0_ref, a1_ref, a2_ref, a3_ref, th_ref, o_ref):
    q0 = a0_ref[...]
    q1 = a1_ref[...]
    q2 = a2_ref[...]
    q3 = a3_ref[...]
    t0 = th_ref[0, 0]
    t1 = th_ref[0, 1]
    t2 = th_ref[0, 2]
    H = FQ // 2
    e0 = jnp.concatenate([q0[:, :H], q1[:, :16]], axis=1)
    o0 = jnp.concatenate([q0[:, H:], q1[:, H:H + 16]], axis=1)
    e1 = jnp.concatenate([q1[:, 16:H], q2[:, :32]], axis=1)
    o1 = jnp.concatenate([q1[:, H + 16:], q2[:, H:H + 32]], axis=1)
    e2 = jnp.concatenate([q2[:, 32:H], q3[:, :H]], axis=1)
    o2 = jnp.concatenate([q2[:, H + 32:], q3[:, H:]], axis=1)
    he = (t0 * jnp.maximum(e0, 0.0) + t1 * jnp.maximum(e1, 0.0)
          + t2 * jnp.maximum(e2, 0.0))
    ho = (t0 * jnp.maximum(o0, 0.0) + t1 * jnp.maximum(o1, 0.0)
          + t2 * jnp.maximum(o2, 0.0))
    h = jnp.concatenate([he, ho], axis=1)
    m = jnp.max(h, axis=1, keepdims=True)
    lse = jnp.log(jnp.sum(jnp.exp(h - m), axis=1, keepdims=True)) + m
    o_ref[...] = h - lse


def _epilogue(accs, theta):
    R = 2000
    return pl.pallas_call(
        _ep_body,
        grid=(N // R,),
        in_specs=[pl.BlockSpec((R, FQ), lambda i: (i, 0))] * 4
        + [pl.BlockSpec(memory_space=pltpu.SMEM)],
        out_specs=pl.BlockSpec((R, D), lambda i: (i, 0)),
        out_shape=jax.ShapeDtypeStruct((N, D), jnp.float32),
    )(*accs, theta)


def kernel(x, edge_index, edge_weight, Ws, bs, thetas):
    w_cat = jnp.concatenate([Ws[-1, 0], Ws[-1, 1], Ws[-1, 2]], axis=1)
    b_cat = bs[-1].reshape(1, 3 * D)
    theta = thetas[-1].reshape(1, 3)
    src = edge_index[0]
    dst = edge_index[1]
    wbits = jax.lax.bitcast_convert_type(edge_weight, jnp.int32)
    meta = jnp.stack(
        [src.reshape(-1, B), dst.reshape(-1, B), wbits.reshape(-1, B)],
        axis=1)

    quarters = _matmul(x, w_cat, b_cat)
    packed = [jax.lax.bitcast_convert_type(
        q.reshape(N, FQ // 2, 2), jnp.int32) for q in quarters]
    a00, a01, a10, a11 = _sc_scatter(*packed, meta)
    out_perm = _epilogue([a00[:N], a01[:N], a10[:N], a11[:N]], theta)
    return jnp.stack([out_perm[:, :D // 2], out_perm[:, D // 2:]],
                     axis=2).reshape(N, D)


# speedup vs baseline: 1.3441x; 1.3441x over previous
import functools

import jax
import jax.numpy as jnp
from jax import lax
from jax.experimental import pallas as pl
from jax.experimental.pallas import tpu as pltpu
from jax.experimental.pallas import tpu_sc as plsc

N = 10000
E = 320000
D = 128
FQ = 96
NTILES = 16
EPT = E // NTILES
B = 80
CPT = EPT // B
RING = 4
MRING = RING + 1
NPAD = 10240
RPT = NPAD // NTILES

_RANGES = {
    (0, 0): ((0, 96, 1),),
    (0, 1): ((0, 32, 1), (32, 96, 2)),
    (1, 0): ((0, 64, 2), (64, 96, 3)),
    (1, 1): ((0, 96, 3),),
}


def _mm_body(x_ref, w_ref, b_ref, o0_ref, o1_ref, o2_ref, o3_ref):
    acc = (jnp.dot(x_ref[...], w_ref[...], preferred_element_type=jnp.float32)
           + b_ref[...])
    o0_ref[...] = acc[:, 0 * FQ:1 * FQ]
    o1_ref[...] = acc[:, 1 * FQ:2 * FQ]
    o2_ref[...] = acc[:, 2 * FQ:3 * FQ]
    o3_ref[...] = acc[:, 3 * FQ:4 * FQ]


def _matmul(x, w, b):
    R = 2000
    return pl.pallas_call(
        _mm_body,
        grid=(N // R,),
        in_specs=[
            pl.BlockSpec((R, D), lambda i: (i, 0)),
            pl.BlockSpec((D, 3 * D), lambda i: (0, 0)),
            pl.BlockSpec((1, 3 * D), lambda i: (0, 0)),
        ],
        out_specs=[pl.BlockSpec((R, FQ), lambda i: (i, 0))] * 4,
        out_shape=[jax.ShapeDtypeStruct((N, FQ), jnp.float32)] * 4,
    )(x, w, b)


def _scale_group(rows_ref, cv, p, base, w1):
    ranges = _RANGES[(cv, p)]
    pw = {1: w1}
    if any(e >= 2 for (_, _, e) in ranges):
        pw[2] = w1 * w1
    if any(e == 3 for (_, _, e) in ranges):
        pw[3] = pw[2] * w1 if 2 in pw else w1 * w1 * w1
    iota = lax.iota(jnp.int32, 16)
    ridx = iota + base

    for (lo, hi, e) in ranges:
        mul = pw[e]

        @plsc.parallel_loop(lo, hi, unroll=16)
        def _(f):
            col = jnp.bitwise_xor(f, iota)
            v = plsc.load_gather(rows_ref, [ridx, col])
            plsc.store_scatter(rows_ref, [ridx, col], v * mul)


def _drain(sem, src_like, dst_like):
    pltpu.make_async_copy(src_like, dst_like, sem).wait()


def _run_pass(cv, p, s, table, out, meta_hbm, meta_v, rows, acc_sh,
              isem, gsems, ssems):
    cbase = s * CPT
    meta_dummy = meta_hbm.at[0]
    rows_dummy = table.at[pl.ds(0, B)]

    def _mwait():
        _drain(isem, meta_dummy, meta_v.at[0])

    def _gfire(k):
        mk = meta_v.at[lax.rem(k, MRING), 0]
        dst = rows.at[pl.ds(lax.rem(k, RING) * B, B)]
        for par in range(2):
            @pl.when(lax.rem(k, 2) == par)
            def _():
                pltpu.async_copy(table.at[mk], dst, gsems[par])

    def _gwait(k):
        for par in range(2):
            @pl.when(lax.rem(k, 2) == par)
            def _():
                _drain(gsems[par], rows_dummy, rows.at[pl.ds(0, B)])

    def _sfire(k):
        sk = rows.at[pl.ds(lax.rem(k, RING) * B, B)]
        dk = acc_sh.at[meta_v.at[lax.rem(k, MRING), 1]]
        for par in range(2):
            @pl.when(lax.rem(k, 2) == par)
            def _():
                pltpu.async_copy(sk, dk, ssems[par], add=True)

    def _swait(k):
        for par in range(2):
            @pl.when(lax.rem(k, 2) == par)
            def _():
                _drain(ssems[par], rows_dummy, rows.at[pl.ds(0, B)])

    pltpu.async_copy(meta_hbm.at[cbase], meta_v.at[0], isem)
    _mwait()
    pltpu.async_copy(meta_hbm.at[cbase + 1], meta_v.at[1], isem)
    _gfire(0)

    @pl.loop(0, CPT)
    def _it(i):
        slot = lax.rem(i, RING)
        mslot = lax.rem(i, MRING)

        @pl.when(i + 1 < CPT)
        def _():
            _mwait()
            _gfire(i + 1)

        @pl.when(i + 2 < CPT)
        def _():
            pltpu.async_copy(meta_hbm.at[cbase + i + 2],
                             meta_v.at[lax.rem(i + 2, MRING)], isem)

        _gwait(i)

        @plsc.parallel_loop(0, B // 16)
        def _grp(g):
            w1 = plsc.bitcast(meta_v[mslot, 2, pl.ds(g * 16, 16)],
                              jnp.float32)
            _scale_group(rows, cv, p, slot * B + g * 16, w1)

        @pl.when(i >= 2)
        def _():
            _swait(i - 2)

        _sfire(i)

    _swait(CPT - 2)
    _swait(CPT - 1)


def _sc_body(q0, q1, q2, q3, meta_hbm,
             o00, o01, o10, o11,
             meta_v, rows, acc_sh, isem, gsem0, gsem1, ssem0, ssem1):
    c = lax.axis_index("c")
    s = lax.axis_index("s")
    tables = ((q0, q1), (q2, q3))
    outs = ((o00, o01), (o10, o11))

    for p in range(2):
        zero16 = jnp.zeros((16,), jnp.float32)

        @pl.loop(0, B)
        def _zrow(i):
            for j in range(FQ // 16):
                rows[i, pl.ds(j * 16, 16)] = zero16

        for k in range(RPT // B):
            pltpu.sync_copy(rows.at[pl.ds(0, B)],
                            acc_sh.at[pl.ds(s * RPT + k * B, B)])
        plsc.subcore_barrier()

        for cv in range(2):
            @pl.when(c == cv)
            def _():
                _run_pass(cv, p, s, tables[cv][p], outs[cv][p], meta_hbm,
                          meta_v, rows, acc_sh, isem, (gsem0, gsem1),
                          (ssem0, ssem1))

        plsc.subcore_barrier()

        for cv in range(2):
            @pl.when(c == cv)
            def _():
                for k in range(RPT // B):
                    r0 = s * RPT + k * B
                    pltpu.sync_copy(acc_sh.at[pl.ds(r0, B)],
                                    outs[cv][p].at[pl.ds(r0, B)])


_sc_scatter = functools.partial(
    pl.kernel,
    out_type=[jax.ShapeDtypeStruct((NPAD, FQ), jnp.float32)] * 4,
    mesh=plsc.VectorSubcoreMesh(core_axis_name="c", subcore_axis_name="s"),
    compiler_params=pltpu.CompilerParams(
        use_tc_tiling_on_sc=False, needs_layout_passes=False),
    scratch_types=[
        pltpu.VMEM((MRING, 3, B), jnp.int32),
        pltpu.VMEM((RING * B, FQ), jnp.float32),
        pltpu.VMEM_SHARED((NPAD, FQ), jnp.float32),
        pltpu.SemaphoreType.DMA,
        pltpu.SemaphoreType.DMA,
        pltpu.SemaphoreType.DMA,
        pltpu.SemaphoreType.DMA,
        pltpu.SemaphoreType.DMA,
    ],
)(_sc_body)


def _ep_body(a0_ref, a1_ref, a2_ref, a3_ref, th_ref, o_ref):
    a0 = a0_ref[...]
    a1 = a1_ref[...]
    a2 = a2_ref[...]
    a3 = a3_ref[...]
    t0 = th_ref[0, 0]
    t1 = th_ref[0, 1]
    t2 = th_ref[0, 2]
    b0 = jnp.concatenate([a0, a1[:, :32]], axis=1)
    b1 = jnp.concatenate([a1[:, 32:], a2[:, :64]], axis=1)
    b2 = jnp.concatenate([a2[:, 64:], a3], axis=1)
    h = (t0 * jnp.maximum(b0, 0.0)
         + t1 * jnp.maximum(b1, 0.0)
         + t2 * jnp.maximum(b2, 0.0))
    m = jnp.max(h, axis=1, keepdims=True)
    lse = jnp.log(jnp.sum(jnp.exp(h - m), axis=1, keepdims=True)) + m
    o_ref[...] = h - lse


def _epilogue(accs, theta):
    R = 2000
    return pl.pallas_call(
        _ep_body,
        grid=(N // R,),
        in_specs=[pl.BlockSpec((R, FQ), lambda i: (i, 0))] * 4
        + [pl.BlockSpec(memory_space=pltpu.SMEM)],
        out_specs=pl.BlockSpec((R, D), lambda i: (i, 0)),
        out_shape=jax.ShapeDtypeStruct((N, D), jnp.float32),
    )(*accs, theta)


def kernel(x, edge_index, edge_weight, Ws, bs, thetas):
    w_cat = jnp.concatenate([Ws[-1, 0], Ws[-1, 1], Ws[-1, 2]], axis=1)
    b_cat = bs[-1].reshape(1, 3 * D)
    theta = thetas[-1].reshape(1, 3)
    src = edge_index[0]
    dst = edge_index[1]
    wbits = jax.lax.bitcast_convert_type(edge_weight, jnp.int32)
    meta = jnp.stack(
        [src.reshape(-1, B), dst.reshape(-1, B), wbits.reshape(-1, B)],
        axis=1)

    quarters = _matmul(x, w_cat, b_cat)
    a00, a01, a10, a11 = _sc_scatter(*quarters, meta)
    return _epilogue([a00[:N], a01[:N], a10[:N], a11[:N]], theta)
